# baseline (speedup 1.0000x reference)
import functools

import jax
import jax.numpy as jnp
from jax import lax
from jax.experimental import pallas as pl
from jax.experimental.pallas import tpu as pltpu
from jax.experimental.pallas import tpu_sc as plsc

_NUM_CODES = 1024
_LATENT_DIM = 12
_CHANNEL_DIM = 64
_ROWS_PER_BLOCK = 128


def _encode_body(x_ref, caug_ref, soft_ref, idx_ref, gidx_ref, cn2_ref):
    @pl.when(pl.program_id(0) == 0)
    def _():
        for l in range(_LATENT_DIM):
            c = caug_ref[l, :, :_CHANNEL_DIM]
            cn2_ref[l, :] = jnp.sum(c * c, axis=1)

    idx_cols = []
    gidx_cols = []
    for l in range(_LATENT_DIM):
        x = x_ref[:, l * _CHANNEL_DIM:(l + 1) * _CHANNEL_DIM]
        caug = caug_ref[l]
        xn2 = jnp.sum(x * x, axis=1, keepdims=True)
        cn2 = cn2_ref[l, :][None, :]
        xc2 = lax.dot_general(
            x + x, caug[:, :_CHANNEL_DIM], (((1,), (1,)), ((), ())),
            preferred_element_type=jnp.float32,
            precision=lax.Precision.HIGHEST)
        d2 = xn2 + cn2 - xc2
        d = jnp.sqrt(jnp.maximum(d2, 0.0))
        e = jnp.exp(-d)
        out = lax.dot_general(
            e, caug, (((1,), (0,)), ((), ())),
            preferred_element_type=jnp.float32)
        s = out[:, _CHANNEL_DIM:_CHANNEL_DIM + 1]
        soft_ref[:, l * _CHANNEL_DIM:(l + 1) * _CHANNEL_DIM] = (
            out[:, :_CHANNEL_DIM] * (1.0 / s))
        idx = jnp.argmin(d2, axis=1)
        idx_cols.append(idx[:, None])
        gidx_cols.append((idx + l * _NUM_CODES)[:, None])
    idx_ref[...] = jnp.concatenate(idx_cols, axis=1)
    gidx_ref[...] = jnp.concatenate(gidx_cols, axis=1)


def _encode(x, codes_aug):
    rows = x.shape[0]
    r = _ROWS_PER_BLOCK
    grid = (rows // r,)
    aug = codes_aug.shape[-1]
    return pl.pallas_call(
        _encode_body,
        grid=grid,
        in_specs=[
            pl.BlockSpec((r, _LATENT_DIM * _CHANNEL_DIM), lambda i: (i, 0)),
            pl.BlockSpec((_LATENT_DIM, _NUM_CODES, aug),
                         lambda i: (0, 0, 0)),
        ],
        scratch_shapes=[pltpu.VMEM((_LATENT_DIM, _NUM_CODES), jnp.float32)],
        compiler_params=pltpu.CompilerParams(
            vmem_limit_bytes=100 * 1024 * 1024),
        out_specs=[
            pl.BlockSpec((r, _LATENT_DIM * _CHANNEL_DIM), lambda i: (i, 0)),
            pl.BlockSpec((r, _LATENT_DIM), lambda i: (i, 0)),
            pl.BlockSpec((r, _LATENT_DIM), lambda i: (i, 0)),
        ],
        out_shape=[
            jax.ShapeDtypeStruct((rows, _LATENT_DIM * _CHANNEL_DIM),
                                 jnp.float32),
            jax.ShapeDtypeStruct((rows, _LATENT_DIM), jnp.int32),
            jax.ShapeDtypeStruct((rows, _LATENT_DIM), jnp.int32),
        ],
    )(x, codes_aug)


_GATHER_CHUNK = 108


def _gather_hard(codes_flat, gidx_rows, total_rows):
    info = plsc.get_sparse_core_info()
    nw = info.num_cores * info.num_subcores
    chunks_total = gidx_rows.shape[0]
    chunks_per_w = chunks_total // nw
    rows_per_w = chunks_per_w * _GATHER_CHUNK

    mesh = plsc.VectorSubcoreMesh(core_axis_name="c", subcore_axis_name="s")

    @functools.partial(
        pl.kernel,
        mesh=mesh,
        out_type=jax.ShapeDtypeStruct((total_rows, _CHANNEL_DIM),
                                      jnp.float32),
        scratch_types=[
            pltpu.VMEM((chunks_per_w, _GATHER_CHUNK), jnp.int32),
            pltpu.VMEM((rows_per_w, _CHANNEL_DIM), jnp.float32),
            pltpu.SemaphoreType.DMA,
        ],
        compiler_params=pltpu.CompilerParams(use_tc_tiling_on_sc=False),
    )
    def k(table_hbm, idx_hbm, out_hbm, idx_v, rows_v, sem):
        wid = lax.axis_index("s") * info.num_cores + lax.axis_index("c")
        pltpu.sync_copy(idx_hbm.at[pl.ds(wid * chunks_per_w, chunks_per_w)],
                        idx_v)
        copies = []
        for j in range(chunks_per_w):
            copies.append(pltpu.async_copy(
                table_hbm.at[idx_v.at[j]],
                rows_v.at[pl.ds(j * _GATHER_CHUNK, _GATHER_CHUNK)],
                sem))
        for cp in copies:
            cp.wait()
        pltpu.sync_copy(rows_v, out_hbm.at[pl.ds(wid * rows_per_w,
                                                 rows_per_w)])

    return k(codes_flat, gidx_rows)


def kernel(z, codes):
    batch, channels, width, height = z.shape
    rows = batch * width * height
    x = jnp.transpose(z, (0, 2, 3, 1)).reshape(rows, channels)
    codes_aug = jnp.concatenate(
        [codes, jnp.ones((_LATENT_DIM, _NUM_CODES, 1), jnp.float32),
         jnp.zeros((_LATENT_DIM, _NUM_CODES, 7), jnp.float32)],
        axis=2)
    soft, idx, gidx = _encode(x, codes_aug)
    codes_flat = codes.reshape(_LATENT_DIM * _NUM_CODES, _CHANNEL_DIM)
    gidx_rows = gidx.reshape(-1, _GATHER_CHUNK)
    hard_flat = _gather_hard(codes_flat, gidx_rows, rows * _LATENT_DIM)
    soft_out = soft.reshape(batch, width, height, channels)
    hard_out = hard_flat.reshape(batch, width, height, channels)
    idx_out = idx.reshape(batch, width, height, _LATENT_DIM)
    return soft_out, hard_out, idx_out


# speedup vs baseline: 1.2215x; 1.2215x over previous
import functools

import jax
import jax.numpy as jnp
from jax import lax
from jax.experimental import pallas as pl
from jax.experimental.pallas import tpu as pltpu
from jax.experimental.pallas import tpu_sc as plsc

_NUM_CODES = 1024
_LATENT_DIM = 12
_CHANNEL_DIM = 64
_ROWS_PER_BLOCK = 256


_AUG = 72


def _encode_body(x_ref, caug_ref, soft_ref, idx_ref, gidx_ref, caug2_ref):
    r = x_ref.shape[0]
    @pl.when(pl.program_id(0) == 0)
    def _():
        caug2_ref[...] = caug_ref[...]
        for l in range(_LATENT_DIM):
            c = caug_ref[l, :, :_CHANNEL_DIM]
            cn2 = jnp.sum(c * c, axis=1, keepdims=True)
            hi = cn2.astype(jnp.bfloat16).astype(jnp.float32)
            caug2_ref[l, :, 65:66] = hi
            caug2_ref[l, :, 66:67] = cn2 - hi

    idx_cols = []
    gidx_cols = []
    ones2 = jnp.ones((r, 2), jnp.float32)
    zeros4 = jnp.zeros((r, _AUG - 68), jnp.float32)
    for l in range(_LATENT_DIM):
        x = x_ref[:, l * _CHANNEL_DIM:(l + 1) * _CHANNEL_DIM]
        caug = caug2_ref[l]
        xn2 = jnp.sum(x * x, axis=1, keepdims=True)
        xa = xn2.astype(jnp.bfloat16).astype(jnp.float32)
        xb = xn2 - xa
        lhs = jnp.concatenate([x * (-2.0), xa, ones2, xb, zeros4], axis=1)
        d2 = lax.dot_general(
            lhs, caug, (((1,), (1,)), ((), ())),
            preferred_element_type=jnp.float32,
            precision=lax.Precision.HIGHEST)
        d = jnp.sqrt(jnp.abs(d2))
        e = jnp.exp(-d)
        out = lax.dot_general(
            e, caug, (((1,), (0,)), ((), ())),
            preferred_element_type=jnp.float32)
        s = out[:, _CHANNEL_DIM:_CHANNEL_DIM + 1]
        soft_ref[:, l * _CHANNEL_DIM:(l + 1) * _CHANNEL_DIM] = (
            out[:, :_CHANNEL_DIM] * (1.0 / s))
        idx = jnp.argmin(d2, axis=1)
        idx_cols.append(idx[:, None])
        gidx_cols.append((idx + l * _NUM_CODES)[:, None])
    idx_ref[...] = jnp.concatenate(idx_cols, axis=1)
    gidx_ref[...] = jnp.concatenate(gidx_cols, axis=1)


def _encode(x, codes_aug):
    rows = x.shape[0]
    r = _ROWS_PER_BLOCK
    grid = (rows // r,)
    aug = codes_aug.shape[-1]
    return pl.pallas_call(
        _encode_body,
        grid=grid,
        in_specs=[
            pl.BlockSpec((r, _LATENT_DIM * _CHANNEL_DIM), lambda i: (i, 0)),
            pl.BlockSpec((_LATENT_DIM, _NUM_CODES, aug),
                         lambda i: (0, 0, 0)),
        ],
        scratch_shapes=[
            pltpu.VMEM((_LATENT_DIM, _NUM_CODES, _AUG), jnp.float32)],
        compiler_params=pltpu.CompilerParams(
            vmem_limit_bytes=100 * 1024 * 1024),
        out_specs=[
            pl.BlockSpec((r, _LATENT_DIM * _CHANNEL_DIM), lambda i: (i, 0)),
            pl.BlockSpec((r, _LATENT_DIM), lambda i: (i, 0)),
            pl.BlockSpec((r, _LATENT_DIM), lambda i: (i, 0)),
        ],
        out_shape=[
            jax.ShapeDtypeStruct((rows, _LATENT_DIM * _CHANNEL_DIM),
                                 jnp.float32),
            jax.ShapeDtypeStruct((rows, _LATENT_DIM), jnp.int32),
            jax.ShapeDtypeStruct((rows, _LATENT_DIM), jnp.int32),
        ],
    )(x, codes_aug)


_GATHER_CHUNK = 108


def _gather_hard(codes_flat, gidx_rows, total_rows):
    info = plsc.get_sparse_core_info()
    nw = info.num_cores * info.num_subcores
    chunks_total = gidx_rows.shape[0]
    chunks_per_w = chunks_total // nw
    rows_per_w = chunks_per_w * _GATHER_CHUNK

    mesh = plsc.VectorSubcoreMesh(core_axis_name="c", subcore_axis_name="s")

    @functools.partial(
        pl.kernel,
        mesh=mesh,
        out_type=jax.ShapeDtypeStruct((total_rows, _CHANNEL_DIM),
                                      jnp.float32),
        scratch_types=[
            pltpu.VMEM((chunks_per_w, _GATHER_CHUNK), jnp.int32),
            pltpu.VMEM((rows_per_w, _CHANNEL_DIM), jnp.float32),
            pltpu.SemaphoreType.DMA,
        ],
        compiler_params=pltpu.CompilerParams(use_tc_tiling_on_sc=False),
    )
    def k(table_hbm, idx_hbm, out_hbm, idx_v, rows_v, sem):
        wid = lax.axis_index("s") * info.num_cores + lax.axis_index("c")
        pltpu.sync_copy(idx_hbm.at[pl.ds(wid * chunks_per_w, chunks_per_w)],
                        idx_v)
        copies = []
        for j in range(chunks_per_w):
            copies.append(pltpu.async_copy(
                table_hbm.at[idx_v.at[j]],
                rows_v.at[pl.ds(j * _GATHER_CHUNK, _GATHER_CHUNK)],
                sem))
        for cp in copies:
            cp.wait()
        pltpu.sync_copy(rows_v, out_hbm.at[pl.ds(wid * rows_per_w,
                                                 rows_per_w)])

    return k(codes_flat, gidx_rows)


def kernel(z, codes):
    batch, channels, width, height = z.shape
    rows = batch * width * height
    x = jnp.transpose(z, (0, 2, 3, 1)).reshape(rows, channels)
    codes_aug = jnp.concatenate(
        [codes, jnp.ones((_LATENT_DIM, _NUM_CODES, 1), jnp.float32),
         jnp.zeros((_LATENT_DIM, _NUM_CODES, 2), jnp.float32),
         jnp.ones((_LATENT_DIM, _NUM_CODES, 1), jnp.float32),
         jnp.zeros((_LATENT_DIM, _NUM_CODES, 4), jnp.float32)],
        axis=2)
    soft, idx, gidx = _encode(x, codes_aug)
    codes_flat = codes.reshape(_LATENT_DIM * _NUM_CODES, _CHANNEL_DIM)
    gidx_rows = gidx.reshape(-1, _GATHER_CHUNK)
    hard_flat = _gather_hard(codes_flat, gidx_rows, rows * _LATENT_DIM)
    soft_out = soft.reshape(batch, width, height, channels)
    hard_out = hard_flat.reshape(batch, width, height, channels)
    idx_out = idx.reshape(batch, width, height, _LATENT_DIM)
    return soft_out, hard_out, idx_out


# speedup vs baseline: 1.2217x; 1.0001x over previous
import functools

import jax
import jax.numpy as jnp
from jax import lax
from jax.experimental import pallas as pl
from jax.experimental.pallas import tpu as pltpu
from jax.experimental.pallas import tpu_sc as plsc

_NUM_CODES = 1024
_LATENT_DIM = 12
_CHANNEL_DIM = 64
_ROWS_PER_BLOCK = 256


_AUG = 72


def _encode_body(x_ref, caug_ref, soft_ref, idx_ref, gidx_ref, caug2_ref):
    r = x_ref.shape[0]
    @pl.when(pl.program_id(0) == 0)
    def _():
        caug2_ref[...] = caug_ref[...]
        for l in range(_LATENT_DIM):
            c = caug_ref[l, :, :_CHANNEL_DIM]
            cn2 = jnp.sum(c * c, axis=1, keepdims=True)
            hi = cn2.astype(jnp.bfloat16).astype(jnp.float32)
            caug2_ref[l, :, 65:66] = hi
            caug2_ref[l, :, 66:67] = cn2 - hi

    idx_cols = []
    gidx_cols = []
    ones2 = jnp.ones((r, 2), jnp.float32)
    zeros4 = jnp.zeros((r, _AUG - 68), jnp.float32)
    for l in range(_LATENT_DIM):
        x = x_ref[:, l * _CHANNEL_DIM:(l + 1) * _CHANNEL_DIM]
        caug = caug2_ref[l]
        xn2 = jnp.sum(x * x, axis=1, keepdims=True)
        xa = xn2.astype(jnp.bfloat16).astype(jnp.float32)
        xb = xn2 - xa
        lhs = jnp.concatenate([x * (-2.0), xa, ones2, xb, zeros4], axis=1)
        d2 = lax.dot_general(
            lhs, caug, (((1,), (1,)), ((), ())),
            preferred_element_type=jnp.float32,
            precision=lax.Precision.HIGHEST)
        d = jnp.sqrt(jnp.abs(d2))
        e = jnp.exp(-d)
        out = lax.dot_general(
            e, caug, (((1,), (0,)), ((), ())),
            preferred_element_type=jnp.float32)
        s = out[:, _CHANNEL_DIM:_CHANNEL_DIM + 1]
        soft_ref[:, l * _CHANNEL_DIM:(l + 1) * _CHANNEL_DIM] = (
            out[:, :_CHANNEL_DIM] * (1.0 / s))
        idx = jnp.argmin(d2, axis=1)
        idx_cols.append(idx[:, None])
        gidx_cols.append((idx + l * _NUM_CODES)[:, None])
    idx_ref[...] = jnp.concatenate(idx_cols, axis=1)
    gidx_ref[...] = jnp.concatenate(gidx_cols, axis=1)


def _encode(x, codes_aug):
    rows = x.shape[0]
    r = _ROWS_PER_BLOCK
    grid = (rows // r,)
    aug = codes_aug.shape[-1]
    return pl.pallas_call(
        _encode_body,
        grid=grid,
        in_specs=[
            pl.BlockSpec((r, _LATENT_DIM * _CHANNEL_DIM), lambda i: (i, 0)),
            pl.BlockSpec((_LATENT_DIM, _NUM_CODES, aug),
                         lambda i: (0, 0, 0)),
        ],
        scratch_shapes=[
            pltpu.VMEM((_LATENT_DIM, _NUM_CODES, _AUG), jnp.float32)],
        out_specs=[
            pl.BlockSpec((r, _LATENT_DIM * _CHANNEL_DIM), lambda i: (i, 0)),
            pl.BlockSpec((r, _LATENT_DIM), lambda i: (i, 0)),
            pl.BlockSpec((r, _LATENT_DIM), lambda i: (i, 0)),
        ],
        out_shape=[
            jax.ShapeDtypeStruct((rows, _LATENT_DIM * _CHANNEL_DIM),
                                 jnp.float32),
            jax.ShapeDtypeStruct((rows, _LATENT_DIM), jnp.int32),
            jax.ShapeDtypeStruct((rows, _LATENT_DIM), jnp.int32),
        ],
    )(x, codes_aug)


_GATHER_CHUNK = 108


def _gather_hard(codes_flat, gidx_rows, total_rows):
    info = plsc.get_sparse_core_info()
    nw = info.num_cores * info.num_subcores
    chunks_total = gidx_rows.shape[0]
    chunks_per_w = chunks_total // nw
    rows_per_w = chunks_per_w * _GATHER_CHUNK

    mesh = plsc.VectorSubcoreMesh(core_axis_name="c", subcore_axis_name="s")

    @functools.partial(
        pl.kernel,
        mesh=mesh,
        out_type=jax.ShapeDtypeStruct((total_rows, _CHANNEL_DIM),
                                      jnp.float32),
        scratch_types=[
            pltpu.VMEM((chunks_per_w, _GATHER_CHUNK), jnp.int32),
            pltpu.VMEM((rows_per_w, _CHANNEL_DIM), jnp.float32),
            pltpu.SemaphoreType.DMA,
        ],
        compiler_params=pltpu.CompilerParams(use_tc_tiling_on_sc=False),
    )
    def k(table_hbm, idx_hbm, out_hbm, idx_v, rows_v, sem):
        wid = lax.axis_index("s") * info.num_cores + lax.axis_index("c")
        pltpu.sync_copy(idx_hbm.at[pl.ds(wid * chunks_per_w, chunks_per_w)],
                        idx_v)
        copies = []
        for j in range(chunks_per_w):
            copies.append(pltpu.async_copy(
                table_hbm.at[idx_v.at[j]],
                rows_v.at[pl.ds(j * _GATHER_CHUNK, _GATHER_CHUNK)],
                sem))
        for cp in copies:
            cp.wait()
        pltpu.sync_copy(rows_v, out_hbm.at[pl.ds(wid * rows_per_w,
                                                 rows_per_w)])

    return k(codes_flat, gidx_rows)


def kernel(z, codes):
    batch, channels, width, height = z.shape
    rows = batch * width * height
    x = jnp.transpose(z, (0, 2, 3, 1)).reshape(rows, channels)
    codes_aug = jnp.concatenate(
        [codes, jnp.ones((_LATENT_DIM, _NUM_CODES, 1), jnp.float32),
         jnp.zeros((_LATENT_DIM, _NUM_CODES, 2), jnp.float32),
         jnp.ones((_LATENT_DIM, _NUM_CODES, 1), jnp.float32),
         jnp.zeros((_LATENT_DIM, _NUM_CODES, 4), jnp.float32)],
        axis=2)
    soft, idx, gidx = _encode(x, codes_aug)
    codes_flat = codes.reshape(_LATENT_DIM * _NUM_CODES, _CHANNEL_DIM)
    gidx_rows = gidx.reshape(-1, _GATHER_CHUNK)
    hard_flat = _gather_hard(codes_flat, gidx_rows, rows * _LATENT_DIM)
    soft_out = soft.reshape(batch, width, height, channels)
    hard_out = hard_flat.reshape(batch, width, height, channels)
    idx_out = idx.reshape(batch, width, height, _LATENT_DIM)
    return soft_out, hard_out, idx_out


# speedup vs baseline: 1.2755x; 1.0441x over previous
import functools

import jax
import jax.numpy as jnp
from jax import lax
from jax.experimental import pallas as pl
from jax.experimental.pallas import tpu as pltpu
from jax.experimental.pallas import tpu_sc as plsc

_NUM_CODES = 1024
_LATENT_DIM = 12
_CHANNEL_DIM = 64
_ROWS_PER_BLOCK = 256


def _encode_body(x_ref, caug_ref, soft_ref, idx_ref, gidx_ref, cn2_ref):
    @pl.when(pl.program_id(0) == 0)
    def _():
        for l in range(_LATENT_DIM):
            c = caug_ref[l, :, :_CHANNEL_DIM]
            cn2_ref[l, :] = jnp.sum(c * c, axis=1)

    idx_cols = []
    gidx_cols = []
    for l in range(_LATENT_DIM):
        x = x_ref[:, l * _CHANNEL_DIM:(l + 1) * _CHANNEL_DIM]
        caug = caug_ref[l]
        xn2 = jnp.sum(x * x, axis=1, keepdims=True)
        cn2 = cn2_ref[l, :][None, :]
        xc2 = lax.dot_general(
            x + x, caug[:, :_CHANNEL_DIM], (((1,), (1,)), ((), ())),
            preferred_element_type=jnp.float32,
            precision=lax.Precision.HIGHEST)
        d2 = xn2 + cn2 - xc2
        d = jnp.sqrt(jnp.maximum(d2, 0.0))
        e = jnp.exp(-d)
        out = lax.dot_general(
            e, caug, (((1,), (0,)), ((), ())),
            preferred_element_type=jnp.float32)
        s = out[:, _CHANNEL_DIM:_CHANNEL_DIM + 1]
        soft_ref[:, l * _CHANNEL_DIM:(l + 1) * _CHANNEL_DIM] = (
            out[:, :_CHANNEL_DIM] * (1.0 / s))
        idx = jnp.argmin(d2, axis=1)
        idx_cols.append(idx[:, None])
        gidx_cols.append((idx + l * _NUM_CODES)[:, None])
    idx_ref[...] = jnp.concatenate(idx_cols, axis=1)
    gidx_ref[...] = jnp.concatenate(gidx_cols, axis=1)


def _encode(x, codes_aug):
    rows = x.shape[0]
    r = _ROWS_PER_BLOCK
    grid = (rows // r,)
    aug = codes_aug.shape[-1]
    return pl.pallas_call(
        _encode_body,
        grid=grid,
        in_specs=[
            pl.BlockSpec((r, _LATENT_DIM * _CHANNEL_DIM), lambda i: (i, 0)),
            pl.BlockSpec((_LATENT_DIM, _NUM_CODES, aug),
                         lambda i: (0, 0, 0)),
        ],
        scratch_shapes=[pltpu.VMEM((_LATENT_DIM, _NUM_CODES), jnp.float32)],
        out_specs=[
            pl.BlockSpec((r, _LATENT_DIM * _CHANNEL_DIM), lambda i: (i, 0)),
            pl.BlockSpec((r, _LATENT_DIM), lambda i: (i, 0)),
            pl.BlockSpec((r, _LATENT_DIM), lambda i: (i, 0)),
        ],
        out_shape=[
            jax.ShapeDtypeStruct((rows, _LATENT_DIM * _CHANNEL_DIM),
                                 jnp.float32),
            jax.ShapeDtypeStruct((rows, _LATENT_DIM), jnp.int32),
            jax.ShapeDtypeStruct((rows, _LATENT_DIM), jnp.int32),
        ],
    )(x, codes_aug)


_GATHER_CHUNK = 108


def _gather_hard(codes_flat, gidx_rows, total_rows):
    info = plsc.get_sparse_core_info()
    nw = info.num_cores * info.num_subcores
    chunks_total = gidx_rows.shape[0]
    chunks_per_w = chunks_total // nw
    rows_per_w = chunks_per_w * _GATHER_CHUNK

    mesh = plsc.VectorSubcoreMesh(core_axis_name="c", subcore_axis_name="s")

    @functools.partial(
        pl.kernel,
        mesh=mesh,
        out_type=jax.ShapeDtypeStruct((total_rows, _CHANNEL_DIM),
                                      jnp.float32),
        scratch_types=[
            pltpu.VMEM((chunks_per_w, _GATHER_CHUNK), jnp.int32),
            pltpu.VMEM((rows_per_w, _CHANNEL_DIM), jnp.float32),
            pltpu.SemaphoreType.DMA,
        ],
        compiler_params=pltpu.CompilerParams(use_tc_tiling_on_sc=False),
    )
    def k(table_hbm, idx_hbm, out_hbm, idx_v, rows_v, sem):
        wid = lax.axis_index("s") * info.num_cores + lax.axis_index("c")
        pltpu.sync_copy(idx_hbm.at[pl.ds(wid * chunks_per_w, chunks_per_w)],
                        idx_v)
        copies = []
        for j in range(chunks_per_w):
            copies.append(pltpu.async_copy(
                table_hbm.at[idx_v.at[j]],
                rows_v.at[pl.ds(j * _GATHER_CHUNK, _GATHER_CHUNK)],
                sem))
        for cp in copies:
            cp.wait()
        pltpu.sync_copy(rows_v, out_hbm.at[pl.ds(wid * rows_per_w,
                                                 rows_per_w)])

    return k(codes_flat, gidx_rows)


def kernel(z, codes):
    batch, channels, width, height = z.shape
    rows = batch * width * height
    x = jnp.transpose(z, (0, 2, 3, 1)).reshape(rows, channels)
    codes_aug = jnp.concatenate(
        [codes, jnp.ones((_LATENT_DIM, _NUM_CODES, 1), jnp.float32),
         jnp.zeros((_LATENT_DIM, _NUM_CODES, 7), jnp.float32)],
        axis=2)
    soft, idx, gidx = _encode(x, codes_aug)
    codes_flat = codes.reshape(_LATENT_DIM * _NUM_CODES, _CHANNEL_DIM)
    gidx_rows = gidx.reshape(-1, _GATHER_CHUNK)
    hard_flat = _gather_hard(codes_flat, gidx_rows, rows * _LATENT_DIM)
    soft_out = soft.reshape(batch, width, height, channels)
    hard_out = hard_flat.reshape(batch, width, height, channels)
    idx_out = idx.reshape(batch, width, height, _LATENT_DIM)
    return soft_out, hard_out, idx_out


# speedup vs baseline: 1.4692x; 1.1518x over previous
import functools

import jax
import jax.numpy as jnp
from jax import lax
from jax.experimental import pallas as pl
from jax.experimental.pallas import tpu as pltpu
from jax.experimental.pallas import tpu_sc as plsc

_NUM_CODES = 1024
_LATENT_DIM = 12
_CHANNEL_DIM = 64
_ROWS_PER_BLOCK = 256


def _encode_body(x_ref, caug_ref, csplit_ref, soft_ref, idx_ref, gidx_ref,
                 cn2_ref):
    @pl.when(pl.program_id(0) == 0)
    def _():
        for l in range(_LATENT_DIM):
            c = caug_ref[l, :, :_CHANNEL_DIM]
            cn2_ref[l, :] = jnp.sum(c * c, axis=1)

    idx_cols = []
    gidx_cols = []
    for l in range(_LATENT_DIM):
        x = x_ref[:, l * _CHANNEL_DIM:(l + 1) * _CHANNEL_DIM]
        caug = caug_ref[l]
        xn2 = jnp.sum(x * x, axis=1, keepdims=True)
        cn2 = cn2_ref[l, :][None, :]
        x2 = x + x
        xh = x2.astype(jnp.bfloat16)
        r1 = x2 - xh.astype(jnp.float32)
        xm = r1.astype(jnp.bfloat16)
        xl = (r1 - xm.astype(jnp.float32)).astype(jnp.bfloat16)
        xs = jnp.concatenate([xh, xh, xm, xh, xm, xl], axis=1)
        xc2 = lax.dot_general(
            xs, csplit_ref[l], (((1,), (1,)), ((), ())),
            preferred_element_type=jnp.float32)
        d2 = xn2 + cn2 - xc2
        d = jnp.sqrt(jnp.maximum(d2, 0.0))
        e = jnp.exp(-d)
        out = lax.dot_general(
            e, caug, (((1,), (0,)), ((), ())),
            preferred_element_type=jnp.float32)
        s = out[:, _CHANNEL_DIM:_CHANNEL_DIM + 1]
        soft_ref[:, l * _CHANNEL_DIM:(l + 1) * _CHANNEL_DIM] = (
            out[:, :_CHANNEL_DIM] * (1.0 / s))
        idx = jnp.argmin(d2, axis=1)
        idx_cols.append(idx[:, None])
        gidx_cols.append((idx + l * _NUM_CODES)[:, None])
    idx_ref[...] = jnp.concatenate(idx_cols, axis=1)
    gidx_ref[...] = jnp.concatenate(gidx_cols, axis=1)


def _encode(x, codes_aug, codes_split):
    rows = x.shape[0]
    r = _ROWS_PER_BLOCK
    grid = (rows // r,)
    aug = codes_aug.shape[-1]
    ksplit = codes_split.shape[-1]
    return pl.pallas_call(
        _encode_body,
        grid=grid,
        in_specs=[
            pl.BlockSpec((r, _LATENT_DIM * _CHANNEL_DIM), lambda i: (i, 0)),
            pl.BlockSpec((_LATENT_DIM, _NUM_CODES, aug),
                         lambda i: (0, 0, 0)),
            pl.BlockSpec((_LATENT_DIM, _NUM_CODES, ksplit),
                         lambda i: (0, 0, 0)),
        ],
        scratch_shapes=[pltpu.VMEM((_LATENT_DIM, _NUM_CODES), jnp.float32)],
        out_specs=[
            pl.BlockSpec((r, _LATENT_DIM * _CHANNEL_DIM), lambda i: (i, 0)),
            pl.BlockSpec((r, _LATENT_DIM), lambda i: (i, 0)),
            pl.BlockSpec((r, _LATENT_DIM), lambda i: (i, 0)),
        ],
        out_shape=[
            jax.ShapeDtypeStruct((rows, _LATENT_DIM * _CHANNEL_DIM),
                                 jnp.float32),
            jax.ShapeDtypeStruct((rows, _LATENT_DIM), jnp.int32),
            jax.ShapeDtypeStruct((rows, _LATENT_DIM), jnp.int32),
        ],
    )(x, codes_aug, codes_split)


_GATHER_CHUNK = 108


def _gather_hard(codes_flat, gidx_rows, total_rows):
    info = plsc.get_sparse_core_info()
    nw = info.num_cores * info.num_subcores
    chunks_total = gidx_rows.shape[0]
    chunks_per_w = chunks_total // nw
    rows_per_w = chunks_per_w * _GATHER_CHUNK

    mesh = plsc.VectorSubcoreMesh(core_axis_name="c", subcore_axis_name="s")

    @functools.partial(
        pl.kernel,
        mesh=mesh,
        out_type=jax.ShapeDtypeStruct((total_rows, _CHANNEL_DIM),
                                      jnp.float32),
        scratch_types=[
            pltpu.VMEM((chunks_per_w, _GATHER_CHUNK), jnp.int32),
            pltpu.VMEM((rows_per_w, _CHANNEL_DIM), jnp.float32),
            pltpu.SemaphoreType.DMA,
        ],
        compiler_params=pltpu.CompilerParams(use_tc_tiling_on_sc=False),
    )
    def k(table_hbm, idx_hbm, out_hbm, idx_v, rows_v, sem):
        wid = lax.axis_index("s") * info.num_cores + lax.axis_index("c")
        pltpu.sync_copy(idx_hbm.at[pl.ds(wid * chunks_per_w, chunks_per_w)],
                        idx_v)
        copies = []
        for j in range(chunks_per_w):
            copies.append(pltpu.async_copy(
                table_hbm.at[idx_v.at[j]],
                rows_v.at[pl.ds(j * _GATHER_CHUNK, _GATHER_CHUNK)],
                sem))
        for cp in copies:
            cp.wait()
        pltpu.sync_copy(rows_v, out_hbm.at[pl.ds(wid * rows_per_w,
                                                 rows_per_w)])

    return k(codes_flat, gidx_rows)


def kernel(z, codes):
    batch, channels, width, height = z.shape
    rows = batch * width * height
    x = jnp.transpose(z, (0, 2, 3, 1)).reshape(rows, channels)
    codes_aug = jnp.concatenate(
        [codes, jnp.ones((_LATENT_DIM, _NUM_CODES, 1), jnp.float32),
         jnp.zeros((_LATENT_DIM, _NUM_CODES, 7), jnp.float32)],
        axis=2)
    ch = codes.astype(jnp.bfloat16)
    r1 = codes - ch.astype(jnp.float32)
    cm = r1.astype(jnp.bfloat16)
    cl = (r1 - cm.astype(jnp.float32)).astype(jnp.bfloat16)
    codes_split = jnp.concatenate([ch, cm, ch, cl, cm, ch], axis=2)
    soft, idx, gidx = _encode(x, codes_aug, codes_split)
    codes_flat = codes.reshape(_LATENT_DIM * _NUM_CODES, _CHANNEL_DIM)
    gidx_rows = gidx.reshape(-1, _GATHER_CHUNK)
    hard_flat = _gather_hard(codes_flat, gidx_rows, rows * _LATENT_DIM)
    soft_out = soft.reshape(batch, width, height, channels)
    hard_out = hard_flat.reshape(batch, width, height, channels)
    idx_out = idx.reshape(batch, width, height, _LATENT_DIM)
    return soft_out, hard_out, idx_out
